# Initial kernel scaffold; baseline (speedup 1.0000x reference)
import jax
import jax.numpy as jnp
from jax.experimental import pallas as pl


def kernel(x, batch_length, params):
    raise NotImplementedError("write your pallas kernel here")


# speedup vs baseline: 5.9187x; 5.9187x over previous
import jax
import jax.numpy as jnp
from jax.experimental import pallas as pl
from jax.experimental.pallas import tpu as pltpu


def _bdot(a, b):
    return jnp.dot(a.astype(jnp.bfloat16), b.astype(jnp.bfloat16),
                   preferred_element_type=jnp.float32)


def _onehot_gather(oh, tbl):
    hi = tbl.astype(jnp.bfloat16).astype(jnp.float32)
    lo = tbl - hi
    return (jnp.dot(oh, hi, preferred_element_type=jnp.float32)
            + jnp.dot(oh, lo, preferred_element_type=jnp.float32))


def _fps(xyz, S):
    B, N, _ = xyz.shape
    x0 = xyz[:, :, 0]
    x1 = xyz[:, :, 1]
    x2 = xyz[:, :, 2]

    def body(x0_ref, x1_ref, x2_ref, out_ref):
        a0 = x0_ref[...]
        a1 = x1_ref[...]
        a2 = x2_ref[...]
        iota = jax.lax.broadcasted_iota(jnp.int32, (B, N), 1).astype(jnp.float32)

        def step(s, carry):
            dist, far = carry
            oh = iota == far
            cx = jnp.sum(jnp.where(oh, a0, 0.0), axis=1, keepdims=True)
            cy = jnp.sum(jnp.where(oh, a1, 0.0), axis=1, keepdims=True)
            cz = jnp.sum(jnp.where(oh, a2, 0.0), axis=1, keepdims=True)
            out_ref[:, pl.ds(s, 1), :] = jnp.concatenate(
                [cx[:, :, None], cy[:, :, None], cz[:, :, None]], axis=2)
            d = ((a0 - cx) ** 2 + (a1 - cy) ** 2) + (a2 - cz) ** 2
            dist = jnp.minimum(dist, d)
            m = jnp.max(dist, axis=1, keepdims=True)
            far = jnp.min(jnp.where(dist == m, iota, float(N)), axis=1,
                          keepdims=True)
            return dist, far

        jax.lax.fori_loop(
            0, S, step,
            (jnp.full((B, N), 1e10, jnp.float32),
             jnp.zeros((B, 1), jnp.float32)))

    return pl.pallas_call(
        body,
        out_shape=jax.ShapeDtypeStruct((B, S, 3), jnp.float32),
    )(x0, x1, x2)


def _sa(xyz, pts, new_xyz, branch_meta, branch_params):
    B, N, C = pts.shape
    S = new_xyz.shape[1]
    xyz_t = jnp.transpose(xyz, (0, 2, 1))
    co_total = sum(p[6].shape[1] for p in branch_params)

    flat_params = [a for p in branch_params for a in p]

    def body(xyz_ref, xyzt_ref, pts_ref, new_ref, *refs):
        out_ref = refs[-1]
        prefs = refs[:-1]
        xyzb = xyz_ref[0]
        xyztb = xyzt_ref[0]
        ptsb = pts_ref[0]
        newb = new_ref[0]
        snew = jnp.sum(newb * newb, axis=1, keepdims=True)
        sxyz = jnp.sum(xyztb * xyztb, axis=0, keepdims=True)
        sq = (snew + sxyz) - 2.0 * _bdot(newb, xyztb)
        iota = jax.lax.broadcasted_iota(jnp.int32, (S, N), 1).astype(jnp.float32)
        outs = []
        for bi, (radius, keff) in enumerate(branch_meta):
            w1a, w1b, b1, w2, b2, w3, b3 = (r[...] for r in
                                            prefs[bi * 7:(bi + 1) * 7])
            h1 = w1a.shape[1]
            tbl = jnp.concatenate(
                [_bdot(ptsb, w1a), xyzb], axis=1)
            key = jnp.where(sq <= radius * radius, iota, float(N))
            acc = None
            first = None
            for k in range(keff):
                m = jnp.min(key, axis=1, keepdims=True)
                if k == 0:
                    first = m
                    sel = m
                else:
                    sel = jnp.where(m >= float(N), first, m)
                key = jnp.where(iota == m, float(3 * N), key)
                oh = (iota == jnp.minimum(sel, float(N - 1))).astype(
                    jnp.float32)
                g = _onehot_gather(oh, tbl)
                gx = g[:, h1:] - newb
                h = g[:, :h1] + _bdot(gx, w1b) + b1
                h = jnp.maximum(h, 0.0)
                h = jnp.maximum(_bdot(h, w2) + b2, 0.0)
                h = jnp.maximum(_bdot(h, w3) + b3, 0.0)
                acc = h if acc is None else jnp.maximum(acc, h)
            outs.append(acc)
        out_ref[0] = jnp.concatenate(outs, axis=1)

    in_specs = [
        pl.BlockSpec((1, N, 3), lambda b: (b, 0, 0)),
        pl.BlockSpec((1, 3, N), lambda b: (b, 0, 0)),
        pl.BlockSpec((1, N, C), lambda b: (b, 0, 0)),
        pl.BlockSpec((1, S, 3), lambda b: (b, 0, 0)),
    ]
    for p in flat_params:
        in_specs.append(pl.BlockSpec(p.shape, lambda b: (0, 0)))

    return pl.pallas_call(
        body,
        grid=(B,),
        in_specs=in_specs,
        out_specs=pl.BlockSpec((1, S, co_total), lambda b: (b, 0, 0)),
        out_shape=jax.ShapeDtypeStruct((B, S, co_total), jnp.float32),
        compiler_params=pltpu.CompilerParams(
            dimension_semantics=("arbitrary",)),
    )(xyz, xyz_t, pts, new_xyz, *flat_params)


def _fp(xyz1, xyz2, pts1, pts2, mlp, final=None):
    B, N1, _ = xyz1.shape
    S2 = xyz2.shape[1]
    xyz2_t = jnp.transpose(xyz2, (0, 2, 1))
    have_p1 = pts1 is not None
    wlist = [a for wb in mlp for a in wb]
    n_relu = len(mlp)
    if final is not None:
        wlist += list(final)
    co = wlist[-2].shape[1]

    def body(*refs):
        out_ref = refs[-1]
        x1 = refs[0][0]
        x2t = refs[1][0]
        i = 2
        p1 = None
        if have_p1:
            p1 = refs[i][0]
            i += 1
        p2 = refs[i][0]
        i += 1
        wrefs = refs[i:-1]
        s1 = jnp.sum(x1 * x1, axis=1, keepdims=True)
        s2 = jnp.sum(x2t * x2t, axis=0, keepdims=True)
        d = (s1 + s2) - 2.0 * _bdot(x1, x2t)
        iota = jax.lax.broadcasted_iota(jnp.int32, (N1, S2), 1).astype(jnp.float32)
        p2_hi = p2.astype(jnp.bfloat16).astype(jnp.float32)
        p2_lo = p2 - p2_hi
        gs = []
        rs = []
        for _ in range(3):
            m = jnp.min(d, axis=1, keepdims=True)
            col = jnp.min(jnp.where(d == m, iota, float(S2)), axis=1,
                          keepdims=True)
            sel = iota == col
            d = jnp.where(sel, 1e30, d)
            dist = jnp.maximum(m, 0.0)
            oh = sel.astype(jnp.float32)
            gs.append(jnp.dot(oh, p2_hi, preferred_element_type=jnp.float32)
                      + jnp.dot(oh, p2_lo, preferred_element_type=jnp.float32))
            rs.append(1.0 / (dist + 1e-8))
        rtot = (rs[0] + rs[1]) + rs[2]
        interp = ((gs[0] * (rs[0] / rtot) + gs[1] * (rs[1] / rtot))
                  + gs[2] * (rs[2] / rtot))
        h = jnp.concatenate([p1, interp], axis=1) if have_p1 else interp
        nw = len(wrefs) // 2
        for li in range(nw):
            w = wrefs[2 * li][...]
            b = wrefs[2 * li + 1][...]
            h = _bdot(h, w) + b
            if li < n_relu:
                h = jnp.maximum(h, 0.0)
        out_ref[0] = h

    operands = [xyz1, xyz2_t]
    in_specs = [
        pl.BlockSpec((1, N1, 3), lambda b: (b, 0, 0)),
        pl.BlockSpec((1, 3, S2), lambda b: (b, 0, 0)),
    ]
    if have_p1:
        operands.append(pts1)
        in_specs.append(
            pl.BlockSpec((1, N1, pts1.shape[2]), lambda b: (b, 0, 0)))
    operands.append(pts2)
    in_specs.append(pl.BlockSpec((1, S2, pts2.shape[2]), lambda b: (b, 0, 0)))
    for w in wlist:
        operands.append(w)
        in_specs.append(pl.BlockSpec(w.shape, lambda b: (0, 0)))

    return pl.pallas_call(
        body,
        grid=(B,),
        in_specs=in_specs,
        out_specs=pl.BlockSpec((1, N1, co), lambda b: (b, 0, 0)),
        out_shape=jax.ShapeDtypeStruct((B, N1, co), jnp.float32),
        compiler_params=pltpu.CompilerParams(
            dimension_semantics=("arbitrary",)),
    )(*operands)


def _prep_sa_params(mlps, C):
    out = []
    for mlp in mlps:
        (w1, b1), (w2, b2), (w3, b3) = mlp
        out.append((w1[:C], w1[C:], b1[None, :], w2, b2[None, :],
                    w3, b3[None, :]))
    return out


_SA_META = {
    'sa1': (0.15, [(0.05, 16), (0.1, 32)]),
    'sa2': (0.25, [(0.1, 16), (0.2, 32)]),
    'sa3': (0.25, [(0.2, 16), (0.4, 32)]),
    'sa4': (0.25, [(0.4, 16), (0.8, 32)]),
}


def _sa_layer(name, xyz, pts, params):
    ratio, meta = _SA_META[name]
    N = xyz.shape[1]
    S = max(1, int(ratio * N))
    meta = [(r, min(k, N)) for (r, k) in meta]
    new_xyz = _fps(xyz, S)
    bp = _prep_sa_params(params[name], pts.shape[2])
    feats = _sa(xyz, pts, new_xyz, meta, bp)
    return new_xyz, feats


def kernel(x, batch_length, params):
    B = int(batch_length.shape[0])
    N = x.shape[0] // B
    xyz0 = x[:, :3].reshape(B, N, 3)
    pts0 = x.reshape(B, N, -1)

    l1x, l1p = _sa_layer('sa1', xyz0, pts0, params)
    l2x, l2p = _sa_layer('sa2', l1x, l1p, params)
    l3x, l3p = _sa_layer('sa3', l2x, l2p, params)
    l4x, l4p = _sa_layer('sa4', l3x, l3p, params)

    def prep_fp(mlp):
        return [(w, b[None, :]) for (w, b) in mlp]

    l3p = _fp(l3x, l4x, l3p, l4p, prep_fp(params['fp4']))
    l2p = _fp(l2x, l3x, l2p, l3p, prep_fp(params['fp3']))
    l1p = _fp(l1x, l2x, l1p, l2p, prep_fp(params['fp2']))
    wc, bc = params['conv1']
    l0p = _fp(xyz0, l1x, None, l1p, prep_fp(params['fp1']),
              final=(wc, bc[None, :]))
    return l0p.reshape(B * N, -1)
